# initial kernel scaffold (unmeasured)
import jax
import jax.numpy as jnp
from jax import lax
from jax.experimental import pallas as pl
from jax.experimental.pallas import tpu as pltpu

N_DEV = 4
N_TOK = 1024
D_IN = 512
D_OUT = 1024
E_LOCAL = 4


def kernel(x, router_W, route_idx, expert_W):
    def body(x_ref, rw_ref, idx_ref, ew_ref, out_ref, comm_ref, send_sems, recv_sems):
        my_pos = lax.axis_index("i")
        left = lax.rem(my_pos + N_DEV - 1, N_DEV)
        right = lax.rem(my_pos + 1, N_DEV)

        barrier_sem = pltpu.get_barrier_semaphore()
        for nbr in (left, right):
            pl.semaphore_signal(
                barrier_sem, inc=1,
                device_id=(nbr,), device_id_type=pl.DeviceIdType.MESH,
            )
        pl.semaphore_wait(barrier_sem, 2)

        xv = x_ref[:, :]
        idx = idx_ref[:, :]
        acc = jnp.zeros((N_TOK, D_OUT), jnp.float32)
        for k in range(E_LOCAL):
            e = my_pos * E_LOCAL + k
            xm = jnp.where(idx == e, xv, 0.0)
            acc = acc + jnp.dot(xm, ew_ref[k], preferred_element_type=jnp.float32)
        comm_ref[0] = acc
        out_ref[:, :] = acc

        for h in range(N_DEV - 1):
            rdma = pltpu.make_async_remote_copy(
                src_ref=comm_ref.at[h],
                dst_ref=comm_ref.at[h + 1],
                send_sem=send_sems.at[h],
                recv_sem=recv_sems.at[h],
                device_id=(right,),
                device_id_type=pl.DeviceIdType.MESH,
            )
            rdma.start()
            rdma.wait()
            out_ref[:, :] = out_ref[:, :] + comm_ref[h + 1]

    return pl.pallas_call(
        body,
        out_shape=jax.ShapeDtypeStruct((N_TOK, D_OUT), jnp.float32),
        in_specs=[
            pl.BlockSpec(memory_space=pltpu.VMEM),
            pl.BlockSpec(memory_space=pltpu.ANY),
            pl.BlockSpec(memory_space=pltpu.VMEM),
            pl.BlockSpec(memory_space=pltpu.VMEM),
        ],
        out_specs=pl.BlockSpec(memory_space=pltpu.VMEM),
        scratch_shapes=[
            pltpu.VMEM((N_DEV, N_TOK, D_OUT), jnp.float32),
            pltpu.SemaphoreType.DMA((N_DEV - 1,)),
            pltpu.SemaphoreType.DMA((N_DEV - 1,)),
        ],
        compiler_params=pltpu.CompilerParams(collective_id=0),
    )(x, router_W, route_idx, expert_W)


# baseline (device time: 158017 ns/iter reference)
import jax
import jax.numpy as jnp
from jax import lax
from jax.experimental import pallas as pl
from jax.experimental.pallas import tpu as pltpu

N_DEV = 4
N_TOK = 1024
D_IN = 512
D_OUT = 1024
E_LOCAL = 4


def kernel(x, router_W, route_idx, expert_W):
    def body(x_ref, rw_ref, idx_ref, ew_ref, out_ref, comm_ref, send_sems, recv_sems):
        my_pos = lax.axis_index("i")
        left = lax.rem(my_pos + N_DEV - 1, N_DEV)
        right = lax.rem(my_pos + 1, N_DEV)

        barrier_sem = pltpu.get_barrier_semaphore()
        for nbr in (left, right):
            pl.semaphore_signal(
                barrier_sem, inc=1,
                device_id=(nbr,), device_id_type=pl.DeviceIdType.MESH,
            )
        pl.semaphore_wait(barrier_sem, 2)

        xv = x_ref[:, :]
        idx = idx_ref[:, :]
        acc = jnp.zeros((N_TOK, D_OUT), jnp.float32)
        for k in range(E_LOCAL):
            e = my_pos * E_LOCAL + k
            xm = jnp.where(idx == e, xv, 0.0)
            acc = acc + jnp.dot(xm, ew_ref[k], preferred_element_type=jnp.float32)
        comm_ref[0] = acc
        out_ref[:, :] = acc

        for h in range(N_DEV - 1):
            rdma = pltpu.make_async_remote_copy(
                src_ref=comm_ref.at[h],
                dst_ref=comm_ref.at[h + 1],
                send_sem=send_sems.at[h],
                recv_sem=recv_sems.at[h],
                device_id=(right,),
                device_id_type=pl.DeviceIdType.MESH,
            )
            rdma.start()
            rdma.wait()
            out_ref[:, :] = out_ref[:, :] + comm_ref[h + 1]

    return pl.pallas_call(
        body,
        out_shape=jax.ShapeDtypeStruct((N_TOK, D_OUT), jnp.float32),
        in_specs=[
            pl.BlockSpec(memory_space=pltpu.VMEM),
            pl.BlockSpec(memory_space=pltpu.VMEM),
            pl.BlockSpec(memory_space=pltpu.VMEM),
            pl.BlockSpec(memory_space=pltpu.VMEM),
        ],
        out_specs=pl.BlockSpec(memory_space=pltpu.VMEM),
        scratch_shapes=[
            pltpu.VMEM((N_DEV, N_TOK, D_OUT), jnp.float32),
            pltpu.SemaphoreType.DMA((N_DEV - 1,)),
            pltpu.SemaphoreType.DMA((N_DEV - 1,)),
        ],
        compiler_params=pltpu.CompilerParams(collective_id=0),
    )(x, router_W, route_idx, expert_W)


# device time: 90722 ns/iter; 1.7418x vs baseline; 1.7418x over previous
import jax
import jax.numpy as jnp
from jax import lax
from jax.experimental import pallas as pl
from jax.experimental.pallas import tpu as pltpu

N_DEV = 4
N_TOK = 1024
D_IN = 512
D_OUT = 1024
E_LOCAL = 4
CHUNK = N_TOK // N_DEV


def kernel(x, router_W, route_idx, expert_W):
    def body(x_ref, rw_ref, idx_ref, ew_ref, out_ref,
             sbuf, rbuf, rs_send_sems, rs_recv_sems, ag_send_sems, ag_recv_sems):
        r = lax.axis_index("i")
        left = lax.rem(r + N_DEV - 1, N_DEV)
        right = lax.rem(r + 1, N_DEV)

        barrier_sem = pltpu.get_barrier_semaphore()
        for nbr in (left, right):
            pl.semaphore_signal(
                barrier_sem, inc=1,
                device_id=(nbr,), device_id_type=pl.DeviceIdType.MESH,
            )
        pl.semaphore_wait(barrier_sem, 2)

        def partial_chunk(c):
            off = c * CHUNK
            xs = x_ref[pl.ds(off, CHUNK), :]
            ids = idx_ref[pl.ds(off, CHUNK), :]
            acc = jnp.zeros((CHUNK, D_OUT), jnp.float32)
            for k in range(E_LOCAL):
                xm = jnp.where(ids == r * E_LOCAL + k, xs, 0.0)
                acc = acc + jnp.dot(xm, ew_ref[k], preferred_element_type=jnp.float32)
            return acc

        sbuf[0] = partial_chunk(r)
        final = None
        for s in range(N_DEV - 1):
            rdma = pltpu.make_async_remote_copy(
                src_ref=sbuf.at[s],
                dst_ref=rbuf.at[s],
                send_sem=rs_send_sems.at[s],
                recv_sem=rs_recv_sems.at[s],
                device_id=(right,),
                device_id_type=pl.DeviceIdType.MESH,
            )
            rdma.start()
            nxt = partial_chunk(lax.rem(r + N_DEV - 1 - s, N_DEV))
            rdma.wait()
            if s < N_DEV - 2:
                sbuf[s + 1] = rbuf[s] + nxt
            else:
                final = rbuf[s] + nxt

        own = lax.rem(r + 1, N_DEV)
        out_ref[pl.ds(own * CHUNK, CHUNK), :] = final

        for t in range(N_DEV - 1):
            src_c = lax.rem(r + 1 - t + N_DEV, N_DEV)
            rdma = pltpu.make_async_remote_copy(
                src_ref=out_ref.at[pl.ds(src_c * CHUNK, CHUNK)],
                dst_ref=out_ref.at[pl.ds(src_c * CHUNK, CHUNK)],
                send_sem=ag_send_sems.at[t],
                recv_sem=ag_recv_sems.at[t],
                device_id=(right,),
                device_id_type=pl.DeviceIdType.MESH,
            )
            rdma.start()
            rdma.wait()

    return pl.pallas_call(
        body,
        out_shape=jax.ShapeDtypeStruct((N_TOK, D_OUT), jnp.float32),
        in_specs=[
            pl.BlockSpec(memory_space=pltpu.VMEM),
            pl.BlockSpec(memory_space=pltpu.VMEM),
            pl.BlockSpec(memory_space=pltpu.VMEM),
            pl.BlockSpec(memory_space=pltpu.VMEM),
        ],
        out_specs=pl.BlockSpec(memory_space=pltpu.VMEM),
        scratch_shapes=[
            pltpu.VMEM((N_DEV - 1, CHUNK, D_OUT), jnp.float32),
            pltpu.VMEM((N_DEV - 1, CHUNK, D_OUT), jnp.float32),
            pltpu.SemaphoreType.DMA((N_DEV - 1,)),
            pltpu.SemaphoreType.DMA((N_DEV - 1,)),
            pltpu.SemaphoreType.DMA((N_DEV - 1,)),
            pltpu.SemaphoreType.DMA((N_DEV - 1,)),
        ],
        compiler_params=pltpu.CompilerParams(collective_id=0),
    )(x, router_W, route_idx, expert_W)


# device time: 57186 ns/iter; 2.7632x vs baseline; 1.5864x over previous
import jax
import jax.numpy as jnp
from jax import lax
from jax.experimental import pallas as pl
from jax.experimental.pallas import tpu as pltpu

N_DEV = 4
N_TOK = 1024
D_IN = 512
D_OUT = 1024
E_LOCAL = 4
CHUNK = N_TOK // N_DEV
H = CHUNK // 2


def kernel(x, router_W, route_idx, expert_W):
    def body(x_ref, rw_ref, idx_ref, ew_ref, out_ref,
             sbuf, rbuf, rs_send_sems, rs_recv_sems, ag_send_sems, ag_recv_sems):
        r = lax.axis_index("i")
        left = lax.rem(r + N_DEV - 1, N_DEV)
        right = lax.rem(r + 1, N_DEV)
        peers = (right, left)

        barrier_sem = pltpu.get_barrier_semaphore()
        for nbr in (left, right):
            pl.semaphore_signal(
                barrier_sem, inc=1,
                device_id=(nbr,), device_id_type=pl.DeviceIdType.MESH,
            )
        pl.semaphore_wait(barrier_sem, 2)

        def partial_half(c, d):
            off = c * CHUNK + d * H
            xs = x_ref[pl.ds(off, H), :]
            ids = idx_ref[pl.ds(off, H), :]
            acc = jnp.zeros((H, D_OUT), jnp.float32)
            for k in range(E_LOCAL):
                xm = jnp.where(ids == r * E_LOCAL + k, xs, 0.0)
                acc = acc + jnp.dot(xm, ew_ref[k], preferred_element_type=jnp.float32)
            return acc

        def rs_recv_chunk(s, d):
            if d == 0:
                return lax.rem(r + N_DEV - 1 - s, N_DEV)
            return lax.rem(r + 1 + s, N_DEV)

        for d in range(2):
            sbuf[d, 0] = partial_half(r, d)
        for s in range(N_DEV - 1):
            rdmas = []
            for d in range(2):
                rdma = pltpu.make_async_remote_copy(
                    src_ref=sbuf.at[d, s],
                    dst_ref=rbuf.at[d, s],
                    send_sem=rs_send_sems.at[d, s],
                    recv_sem=rs_recv_sems.at[d, s],
                    device_id=(peers[d],),
                    device_id_type=pl.DeviceIdType.MESH,
                )
                rdma.start()
                rdmas.append(rdma)
            nxt = [partial_half(rs_recv_chunk(s, d), d) for d in range(2)]
            for d in range(2):
                rdmas[d].wait()
                val = rbuf[d, s] + nxt[d]
                if s < N_DEV - 2:
                    sbuf[d, s + 1] = val
                else:
                    own = rs_recv_chunk(s, d)
                    out_ref[pl.ds(own * CHUNK + d * H, H), :] = val

        for t in range(N_DEV - 1):
            rdmas = []
            for d in range(2):
                if d == 0:
                    c = lax.rem(r + 1 - t + N_DEV, N_DEV)
                else:
                    c = lax.rem(r - 1 + t + N_DEV, N_DEV)
                off = c * CHUNK + d * H
                rdma = pltpu.make_async_remote_copy(
                    src_ref=out_ref.at[pl.ds(off, H)],
                    dst_ref=out_ref.at[pl.ds(off, H)],
                    send_sem=ag_send_sems.at[d, t],
                    recv_sem=ag_recv_sems.at[d, t],
                    device_id=(peers[d],),
                    device_id_type=pl.DeviceIdType.MESH,
                )
                rdma.start()
                rdmas.append(rdma)
            for rdma in rdmas:
                rdma.wait()

    return pl.pallas_call(
        body,
        out_shape=jax.ShapeDtypeStruct((N_TOK, D_OUT), jnp.float32),
        in_specs=[
            pl.BlockSpec(memory_space=pltpu.VMEM),
            pl.BlockSpec(memory_space=pltpu.VMEM),
            pl.BlockSpec(memory_space=pltpu.VMEM),
            pl.BlockSpec(memory_space=pltpu.VMEM),
        ],
        out_specs=pl.BlockSpec(memory_space=pltpu.VMEM),
        scratch_shapes=[
            pltpu.VMEM((2, N_DEV - 1, H, D_OUT), jnp.float32),
            pltpu.VMEM((2, N_DEV - 1, H, D_OUT), jnp.float32),
            pltpu.SemaphoreType.DMA((2, N_DEV - 1)),
            pltpu.SemaphoreType.DMA((2, N_DEV - 1)),
            pltpu.SemaphoreType.DMA((2, N_DEV - 1)),
            pltpu.SemaphoreType.DMA((2, N_DEV - 1)),
        ],
        compiler_params=pltpu.CompilerParams(collective_id=0),
    )(x, router_W, route_idx, expert_W)


# device time: 40375 ns/iter; 3.9137x vs baseline; 1.4164x over previous
import jax
import jax.numpy as jnp
from jax import lax
from jax.experimental import pallas as pl
from jax.experimental.pallas import tpu as pltpu

N_DEV = 4
N_TOK = 1024
D_IN = 512
D_OUT = 1024
E_LOCAL = 4
CHUNK = N_TOK // N_DEV
H = CHUNK // 2


def kernel(x, router_W, route_idx, expert_W):
    def body(x_ref, rw_ref, idx_ref, ew_ref, out_ref,
             xb, ewb, obuf, sbuf, rbuf,
             rs_send_sems, rs_recv_sems, ag_send_sems, ag_recv_sems):
        r = lax.axis_index("i")
        left = lax.rem(r + N_DEV - 1, N_DEV)
        right = lax.rem(r + 1, N_DEV)
        peers = (right, left)

        barrier_sem = pltpu.get_barrier_semaphore()
        for nbr in (left, right):
            pl.semaphore_signal(
                barrier_sem, inc=1,
                device_id=(nbr,), device_id_type=pl.DeviceIdType.MESH,
            )
        pl.semaphore_wait(barrier_sem, 2)

        xb[:, :] = x_ref[:, :].astype(jnp.bfloat16)
        ewb[...] = ew_ref[...].astype(jnp.bfloat16)

        def partial_half(c, d):
            off = c * CHUNK + d * H
            xs = xb[pl.ds(off, H), :]
            ids = idx_ref[pl.ds(off, H), :]
            acc = jnp.zeros((H, D_OUT), jnp.float32)
            for k in range(E_LOCAL):
                xm = jnp.where(ids == r * E_LOCAL + k, xs, jnp.bfloat16(0))
                acc = acc + jnp.dot(xm, ewb[k], preferred_element_type=jnp.float32)
            return acc.astype(jnp.bfloat16)

        def rs_recv_chunk(s, d):
            if d == 0:
                return lax.rem(r + N_DEV - 1 - s, N_DEV)
            return lax.rem(r + 1 + s, N_DEV)

        for d in range(2):
            sbuf[d, 0] = partial_half(r, d)
        for s in range(N_DEV - 1):
            rdmas = []
            for d in range(2):
                rdma = pltpu.make_async_remote_copy(
                    src_ref=sbuf.at[d, s],
                    dst_ref=rbuf.at[d, s],
                    send_sem=rs_send_sems.at[d, s],
                    recv_sem=rs_recv_sems.at[d, s],
                    device_id=(peers[d],),
                    device_id_type=pl.DeviceIdType.MESH,
                )
                rdma.start()
                rdmas.append(rdma)
            nxt = [partial_half(rs_recv_chunk(s, d), d) for d in range(2)]
            for d in range(2):
                rdmas[d].wait()
                val = rbuf[d, s] + nxt[d]
                if s < N_DEV - 2:
                    sbuf[d, s + 1] = val
                else:
                    own = rs_recv_chunk(s, d)
                    obuf[pl.ds(own * CHUNK + d * H, H), :] = val

        for t in range(N_DEV - 1):
            rdmas = []
            for d in range(2):
                if d == 0:
                    c = lax.rem(r + 1 - t + N_DEV, N_DEV)
                else:
                    c = lax.rem(r - 1 + t + N_DEV, N_DEV)
                off = c * CHUNK + d * H
                rdma = pltpu.make_async_remote_copy(
                    src_ref=obuf.at[pl.ds(off, H)],
                    dst_ref=obuf.at[pl.ds(off, H)],
                    send_sem=ag_send_sems.at[d, t],
                    recv_sem=ag_recv_sems.at[d, t],
                    device_id=(peers[d],),
                    device_id_type=pl.DeviceIdType.MESH,
                )
                rdma.start()
                rdmas.append(rdma)
            for rdma in rdmas:
                rdma.wait()

        out_ref[:, :] = obuf[:, :].astype(jnp.float32)

    return pl.pallas_call(
        body,
        out_shape=jax.ShapeDtypeStruct((N_TOK, D_OUT), jnp.float32),
        in_specs=[
            pl.BlockSpec(memory_space=pltpu.VMEM),
            pl.BlockSpec(memory_space=pltpu.VMEM),
            pl.BlockSpec(memory_space=pltpu.VMEM),
            pl.BlockSpec(memory_space=pltpu.VMEM),
        ],
        out_specs=pl.BlockSpec(memory_space=pltpu.VMEM),
        scratch_shapes=[
            pltpu.VMEM((N_TOK, D_IN), jnp.bfloat16),
            pltpu.VMEM((E_LOCAL, D_IN, D_OUT), jnp.bfloat16),
            pltpu.VMEM((N_TOK, D_OUT), jnp.bfloat16),
            pltpu.VMEM((2, N_DEV - 1, H, D_OUT), jnp.bfloat16),
            pltpu.VMEM((2, N_DEV - 1, H, D_OUT), jnp.bfloat16),
            pltpu.SemaphoreType.DMA((2, N_DEV - 1)),
            pltpu.SemaphoreType.DMA((2, N_DEV - 1)),
            pltpu.SemaphoreType.DMA((2, N_DEV - 1)),
            pltpu.SemaphoreType.DMA((2, N_DEV - 1)),
        ],
        compiler_params=pltpu.CompilerParams(collective_id=0),
    )(x, router_W, route_idx, expert_W)


# device time: 39764 ns/iter; 3.9739x vs baseline; 1.0154x over previous
import jax
import jax.numpy as jnp
from jax import lax
from jax.experimental import pallas as pl
from jax.experimental.pallas import tpu as pltpu

N_DEV = 4
N_TOK = 1024
D_IN = 512
D_OUT = 1024
E_LOCAL = 4
CHUNK = N_TOK // N_DEV
H = CHUNK // 2


def kernel(x, router_W, route_idx, expert_W):
    def body(x_ref, rw_ref, idx_ref, ew_ref, out_ref,
             xb, ewb, obuf, sbuf, rbuf,
             rs_send_sems, rs_recv_sems, ag_send_sems, ag_recv_sems):
        r = lax.axis_index("i")
        left = lax.rem(r + N_DEV - 1, N_DEV)
        right = lax.rem(r + 1, N_DEV)
        diag = lax.rem(r + 2, N_DEV)
        peers = (right, left)

        barrier_sem = pltpu.get_barrier_semaphore()
        for nbr in (left, right):
            pl.semaphore_signal(
                barrier_sem, inc=1,
                device_id=(nbr,), device_id_type=pl.DeviceIdType.MESH,
            )

        xb[:, :] = x_ref[:, :].astype(jnp.bfloat16)
        ewb[...] = ew_ref[...].astype(jnp.bfloat16)

        def partial_half(c, d):
            off = c * CHUNK + d * H
            xs = xb[pl.ds(off, H), :]
            ids = idx_ref[pl.ds(off, H), :]
            acc = jnp.zeros((H, D_OUT), jnp.float32)
            for k in range(E_LOCAL):
                xm = jnp.where(ids == r * E_LOCAL + k, xs, jnp.bfloat16(0))
                acc = acc + jnp.dot(xm, ewb[k], preferred_element_type=jnp.float32)
            return acc.astype(jnp.bfloat16)

        def rs_recv_chunk(s, d):
            if d == 0:
                return lax.rem(r + N_DEV - 1 - s, N_DEV)
            return lax.rem(r + 1 + s, N_DEV)

        def own_chunk(d):
            return lax.rem(r + 1, N_DEV) if d == 0 else lax.rem(r + N_DEV - 1, N_DEV)

        for d in range(2):
            sbuf[d, 0] = partial_half(r, d)

        pl.semaphore_wait(barrier_sem, 2)

        deferred = []

        for s in range(N_DEV - 1):
            rdmas = []
            for d in range(2):
                rdma = pltpu.make_async_remote_copy(
                    src_ref=sbuf.at[d, s],
                    dst_ref=rbuf.at[d, s],
                    send_sem=rs_send_sems.at[d, s],
                    recv_sem=rs_recv_sems.at[d, s],
                    device_id=(peers[d],),
                    device_id_type=pl.DeviceIdType.MESH,
                )
                rdma.start()
                rdmas.append(rdma)
                deferred.append(rdma)
            nxt = [partial_half(rs_recv_chunk(s, d), d) for d in range(2)]
            for d in range(2):
                rdmas[d].wait_recv()
                val = rbuf[d, s] + nxt[d]
                if s < N_DEV - 2:
                    sbuf[d, s + 1] = val
                else:
                    obuf[pl.ds(own_chunk(d) * CHUNK + d * H, H), :] = val

        for d in range(2):
            off = own_chunk(d) * CHUNK + d * H
            for j, tgt in enumerate((right, left, diag)):
                rdma = pltpu.make_async_remote_copy(
                    src_ref=obuf.at[pl.ds(off, H)],
                    dst_ref=obuf.at[pl.ds(off, H)],
                    send_sem=ag_send_sems.at[d, j],
                    recv_sem=ag_recv_sems.at[d, j],
                    device_id=(tgt,),
                    device_id_type=pl.DeviceIdType.MESH,
                )
                rdma.start()
                deferred.append(rdma)

        for d in range(2):
            off = own_chunk(d) * CHUNK + d * H
            out_ref[pl.ds(off, H), :] = obuf[pl.ds(off, H), :].astype(jnp.float32)

        senders = (left, right, diag)
        for j in range(3):
            for d in range(2):
                c = lax.rem(senders[j] + (1 if d == 0 else N_DEV - 1), N_DEV)
                off = c * CHUNK + d * H
                recv = pltpu.make_async_remote_copy(
                    src_ref=obuf.at[pl.ds(off, H)],
                    dst_ref=obuf.at[pl.ds(off, H)],
                    send_sem=ag_send_sems.at[d, j],
                    recv_sem=ag_recv_sems.at[d, j],
                    device_id=(senders[j],),
                    device_id_type=pl.DeviceIdType.MESH,
                )
                recv.wait_recv()
                out_ref[pl.ds(off, H), :] = obuf[pl.ds(off, H), :].astype(jnp.float32)

        for rdma in deferred:
            rdma.wait_send()

    return pl.pallas_call(
        body,
        out_shape=jax.ShapeDtypeStruct((N_TOK, D_OUT), jnp.float32),
        in_specs=[
            pl.BlockSpec(memory_space=pltpu.VMEM),
            pl.BlockSpec(memory_space=pltpu.VMEM),
            pl.BlockSpec(memory_space=pltpu.VMEM),
            pl.BlockSpec(memory_space=pltpu.VMEM),
        ],
        out_specs=pl.BlockSpec(memory_space=pltpu.VMEM),
        scratch_shapes=[
            pltpu.VMEM((N_TOK, D_IN), jnp.bfloat16),
            pltpu.VMEM((E_LOCAL, D_IN, D_OUT), jnp.bfloat16),
            pltpu.VMEM((N_TOK, D_OUT), jnp.bfloat16),
            pltpu.VMEM((2, N_DEV - 1, H, D_OUT), jnp.bfloat16),
            pltpu.VMEM((2, N_DEV - 1, H, D_OUT), jnp.bfloat16),
            pltpu.SemaphoreType.DMA((2, N_DEV - 1)),
            pltpu.SemaphoreType.DMA((2, N_DEV - 1)),
            pltpu.SemaphoreType.DMA((2, 3)),
            pltpu.SemaphoreType.DMA((2, 3)),
        ],
        compiler_params=pltpu.CompilerParams(collective_id=0),
    )(x, router_W, route_idx, expert_W)
